# baseline (speedup 1.0000x reference)
import functools

import jax
import jax.numpy as jnp
from jax import lax
from jax.experimental import pallas as pl
from jax.experimental.pallas import tpu as pltpu
from jax.experimental.pallas import tpu_sc as plsc

Q = 7
VOCAB = 1025
D = 1024
DP = D // 2
T = 16384
NC, NS = 2, 16
NW = NC * NS
TPW = T // NW
C = 8
NCH = TPW // C
NGRP = NCH // 2
LANES = 16
GPR = DP // LANES

def _make_embed_call():
    mesh = plsc.VectorSubcoreMesh(core_axis_name="c", subcore_axis_name="s")

    @functools.partial(
        pl.kernel,
        out_type=jax.ShapeDtypeStruct((T, D), jnp.float32),
        mesh=mesh,
        scratch_types=[
            pltpu.VMEM((Q, TPW), jnp.int32),
            pltpu.VMEM((4, Q, C, DP), jnp.int32),
            pltpu.VMEM((C, D), jnp.float32),
            pltpu.SemaphoreType.DMA((4,)),
            pltpu.SemaphoreType.DMA,
        ],
    )
    def embed(w_hbm, idx_hbm, out_hbm,
              idx_v, rings, stage, gsems, osem):

        wid = lax.axis_index("s") * NC + lax.axis_index("c")
        base = wid * TPW

        pltpu.sync_copy(idx_hbm.at[wid], idx_v)

        def gather_desc(k, cc):
            rp = jnp.bitwise_and(cc, 3)
            idx_sl = idx_v.at[k, pl.ds(cc * C, C)]
            return pltpu.make_async_copy(
                w_hbm.at[idx_sl], rings.at[rp, k], gsems.at[rp])

        def out_desc(cc):
            return pltpu.make_async_copy(
                stage, out_hbm.at[pl.ds(base + cc * C, C)], osem)

        def accum(rp):
            hi_mask = jnp.int32(-65536)

            @plsc.parallel_loop(0, C * GPR, 1, unroll=4)
            def _(i):
                r = lax.shift_right_logical(i, 5)
                g = jnp.bitwise_and(i, GPR - 1)
                colp = pl.multiple_of(lax.shift_left(g, 4), LANES)
                slp = pl.ds(colp, LANES)
                lo = None
                hi = None
                for k in range(Q):
                    v = rings[rp, k, r, slp]
                    vlo = lax.bitcast_convert_type(lax.shift_left(v, 16), jnp.float32)
                    vhi = lax.bitcast_convert_type(jnp.bitwise_and(v, hi_mask), jnp.float32)
                    lo = vlo if lo is None else lo + vlo
                    hi = vhi if hi is None else hi + vhi
                stage[r, pl.ds(colp, LANES)] = lo
                stage[r, pl.ds(colp + DP, LANES)] = hi

        def issue_chunk(cc):
            for k in range(Q):
                gather_desc(k, cc).start()

        issue_chunk(0)
        issue_chunk(1)
        issue_chunk(2)

        def chunk_body(cc, _):
            @pl.when(cc + 3 < NCH)
            def _():
                issue_chunk(cc + 3)
            for k in range(Q):
                gather_desc(k, cc).wait()
            @pl.when(cc >= 1)
            def _():
                out_desc(cc).wait()
            accum(jnp.bitwise_and(cc, 3))
            out_desc(cc).start()
            return 0

        lax.fori_loop(0, NCH, chunk_body, 0)

        out_desc(NCH - 1).wait()

    return embed


_embed = _make_embed_call()


def kernel(xi, W):
    idx = xi[:, :Q].astype(jnp.int32) + (jnp.arange(Q, dtype=jnp.int32) * VOCAB)[None, :]
    idx_all = idx.T.reshape(Q, NW, TPW).transpose(1, 0, 2)
    lo16 = lax.bitcast_convert_type(W[..., :DP].astype(jnp.bfloat16), jnp.uint16)
    hi16 = lax.bitcast_convert_type(W[..., DP:].astype(jnp.bfloat16), jnp.uint16)
    w32 = lo16.astype(jnp.uint32) | (hi16.astype(jnp.uint32) << 16)
    w_packed = lax.bitcast_convert_type(w32, jnp.int32).reshape(W.shape[0] * VOCAB, DP)
    return _embed(w_packed, idx_all)


# speedup vs baseline: 1.0580x; 1.0580x over previous
import functools

import jax
import jax.numpy as jnp
from jax import lax
from jax.experimental import pallas as pl
from jax.experimental.pallas import tpu as pltpu
from jax.experimental.pallas import tpu_sc as plsc

Q = 7
VOCAB = 1025
D = 1024
DP = D // 2
T = 16384
NC, NS = 2, 16
NW = NC * NS
TPW = T // NW
C = 8
NCH = TPW // C
NGRP = NCH // 2
LANES = 16
GPR = DP // LANES

def _make_embed_call():
    mesh = plsc.VectorSubcoreMesh(core_axis_name="c", subcore_axis_name="s")

    @functools.partial(
        pl.kernel,
        out_type=jax.ShapeDtypeStruct((T, D), jnp.float32),
        mesh=mesh,
        scratch_types=[
            pltpu.VMEM((Q, TPW), jnp.int32),
            pltpu.VMEM((3, Q, C, DP), jnp.int32),
            pltpu.VMEM((C, D), jnp.float32),
            pltpu.VMEM((C, D), jnp.float32),
            pltpu.SemaphoreType.DMA((3,)),
            pltpu.SemaphoreType.DMA,
            pltpu.SemaphoreType.DMA,
        ],
    )
    def embed(w_hbm, idx_hbm, out_hbm,
              idx_v, rings, stage0, stage1, gsems, osem0, osem1):
        stages = (stage0, stage1)
        osems = (osem0, osem1)

        wid = lax.axis_index("s") * NC + lax.axis_index("c")
        base = wid * TPW

        pltpu.sync_copy(idx_hbm.at[wid], idx_v)

        def gather_desc(k, cc):
            rp = lax.rem(cc, 3)
            idx_sl = idx_v.at[k, pl.ds(cc * C, C)]
            return pltpu.make_async_copy(
                w_hbm.at[idx_sl], rings.at[rp, k], gsems.at[rp])

        def out_desc(cc, p):
            return pltpu.make_async_copy(
                stages[p], out_hbm.at[pl.ds(base + cc * C, C)], osems[p])

        def accum(p, rp):
            @plsc.parallel_loop(0, C * GPR, 1, unroll=8)
            def _(i):
                r = lax.shift_right_logical(i, 5)
                g = jnp.bitwise_and(i, GPR - 1)
                colp = pl.multiple_of(lax.shift_left(g, 4), LANES)
                slp = pl.ds(colp, LANES)
                lo = None
                hi = None
                for k in range(Q):
                    v = rings[rp, k, r, slp]
                    vlo = lax.bitcast_convert_type(lax.shift_left(v, 16), jnp.float32)
                    vhi = lax.bitcast_convert_type(v, jnp.float32)
                    lo = vlo if lo is None else lo + vlo
                    hi = vhi if hi is None else hi + vhi
                stages[p][r, pl.ds(colp, LANES)] = lo
                stages[p][r, pl.ds(colp + DP, LANES)] = hi

        def issue_chunk(cc):
            for k in range(Q):
                gather_desc(k, cc).start()

        issue_chunk(0)
        issue_chunk(1)

        def group_body(g, _):
            c0 = g * 2
            for b in range(2):
                cc = c0 + b
                @pl.when(cc + 2 < NCH)
                def _():
                    issue_chunk(cc + 2)
                for k in range(Q):
                    gather_desc(k, cc).wait()
                @pl.when(g >= 1)
                def _():
                    out_desc(cc, b).wait()
                accum(b, lax.rem(cc, 3))
                out_desc(cc, b).start()
            return 0

        lax.fori_loop(0, NGRP, group_body, 0)

        out_desc(NCH - 2, 0).wait()
        out_desc(NCH - 1, 1).wait()

    return embed


_embed = _make_embed_call()


def kernel(xi, W):
    idx = xi[:, :Q].astype(jnp.int32) + (jnp.arange(Q, dtype=jnp.int32) * VOCAB)[None, :]
    idx_all = idx.T.reshape(Q, NW, TPW).transpose(1, 0, 2)
    lo16 = lax.bitcast_convert_type(W[..., :DP].astype(jnp.bfloat16), jnp.uint16)
    hi16 = lax.bitcast_convert_type(W[..., DP:].astype(jnp.bfloat16), jnp.uint16)
    w32 = lo16.astype(jnp.uint32) | (hi16.astype(jnp.uint32) << 16)
    w_packed = lax.bitcast_convert_type(w32, jnp.int32).reshape(W.shape[0] * VOCAB, DP)
    return _embed(w_packed, idx_all)
